# baseline (speedup 1.0000x reference)
import functools

import jax
import jax.numpy as jnp
from jax import lax
from jax.experimental import pallas as pl
from jax.experimental.pallas import tpu as pltpu
from jax.experimental.pallas import tpu_sc as plsc

L = 16
NC = 2
NS = 16
WSZ = 1 << 20
PBITS = 14


def _t_body(in_ref, out_ref):
    out_ref[...] = in_ref[...].T


def _to_rowmajor(xT):
    _, N = xT.shape
    bc = 512
    grid = (N + bc - 1) // bc
    return pl.pallas_call(
        _t_body,
        grid=(grid,),
        in_specs=[pl.BlockSpec((L, bc), lambda i: (0, i))],
        out_specs=pl.BlockSpec((bc, L), lambda i: (i, 0)),
        out_shape=jax.ShapeDtypeStruct((N, L), jnp.float32),
    )(xT)


def _to_native(x):
    N, _ = x.shape
    bc = 512
    grid = (N + bc - 1) // bc
    return pl.pallas_call(
        _t_body,
        grid=(grid,),
        in_specs=[pl.BlockSpec((bc, L), lambda i: (i, 0))],
        out_specs=pl.BlockSpec((L, bc), lambda i: (0, i)),
        out_shape=jax.ShapeDtypeStruct((L, N), jnp.float32),
    )(x)


def _scatter_body(out_ref, idx_ref, val_ref,
                  idx_d, sidx, candv, wbuf, idx_s, wsel, eidx, gidx, vrows,
                  zbuf, w_tab, sem0, sem1):
    c = lax.axis_index("c")
    s = lax.axis_index("s")
    M = out_ref.shape[0] // L
    B = idx_ref.shape[0]
    dpw = B // NS
    spw = B // (NC * NS)
    drows = dpw // 128
    srows = spw // 128
    iota = lax.iota(jnp.int32, L)

    def _zfill(i, _):
        zbuf[pl.ds(i * L, L)] = jnp.zeros((L,), jnp.int32)
        return 0
    lax.fori_loop(0, zbuf.shape[0] // L, _zfill, 0)
    zpw = WSZ // NS
    nz = zpw // zbuf.shape[0]
    zd = [pltpu.async_copy(zbuf, w_tab.at[pl.ds(s * zpw + t * zbuf.shape[0],
                                                zbuf.shape[0])], sem0)
          for t in range(nz)]
    dbase = s * dpw
    sbase = c * (NS * spw) + s * spw
    ld = [pltpu.async_copy(idx_ref.at[pl.ds(dbase + j * 128, 128)],
                           idx_d.at[j], sem1) for j in range(drows)]
    li = [pltpu.async_copy(idx_ref.at[pl.ds(sbase + j * 128, 128)],
                           idx_s.at[j], sem1) for j in range(srows)]
    for d in zd + ld + li:
        d.wait()

    def _build_eidx(src, dst, j):
        def _grp(k, _):
            v = src[j, pl.ds(k * L, L)]
            for l in range(L):
                row = j * 128 + k * L + l
                dst[(row * L) // 128, pl.ds((row * L) % 128, L)] = \
                    v[l] * L + iota
            return 0
        lax.fori_loop(0, 128 // L, _grp, 0)

    for j in range(srows):
        _build_eidx(idx_s, eidx, j)

    plsc.subcore_barrier()

    dummy = (M + iota).astype(jnp.int32)
    for b in range(PBITS - 1, -1, -1):
        def _prep(g, _, b=b):
            j = g // (128 // L)
            k = g % (128 // L)
            pos = dbase + g * L + iota
            w = wbuf[j, pl.ds(k * L, L)] if b < PBITS - 1 else pos * 0
            alive = lax.shift_right_logical(pos, b + 1) == \
                lax.shift_right_logical(w, b + 1)
            writer = jnp.logical_and(
                alive, (lax.shift_right_logical(pos, b) & 1) > 0)
            pref = lax.shift_left(lax.shift_right_logical(pos, b), b)
            candv[j, pl.ds(k * L, L)] = pref
            sidx[j, pl.ds(k * L, L)] = jnp.where(
                writer, idx_d[j, pl.ds(k * L, L)], dummy)
            return 0
        lax.fori_loop(0, dpw // L, _prep, 0)
        sc = [pltpu.async_copy(candv.at[j], w_tab.at[sidx.at[j]], sem0)
              for j in range(drows)]
        for d in sc:
            d.wait()
        plsc.subcore_barrier()
        if b > 0:
            ga = [pltpu.async_copy(w_tab.at[idx_d.at[j]], wbuf.at[j], sem0)
                  for j in range(drows)]
            for d in ga:
                d.wait()
            plsc.subcore_barrier()

    gw = [pltpu.async_copy(w_tab.at[idx_s.at[j]], wsel.at[j], sem0)
          for j in range(srows)]
    for d in gw:
        d.wait()
    for j in range(srows):
        _build_eidx(wsel, gidx, j)
    ng = spw * L // 128
    gv = [pltpu.async_copy(val_ref.at[gidx.at[g]], vrows.at[g], sem1)
          for g in range(ng)]
    for d in gv:
        d.wait()
    st = [pltpu.async_copy(vrows.at[g], out_ref.at[eidx.at[g]], sem0)
          for g in range(ng)]
    for d in st:
        d.wait()


def kernel(mem, idx, val):
    M, D = mem.shape
    B = idx.shape[0]
    assert D == L and B == 1 << PBITS and M + L <= WSZ

    mesh = plsc.VectorSubcoreMesh(core_axis_name="c", subcore_axis_name="s")
    scatter = functools.partial(
        pl.kernel,
        out_type=(),
        mesh=mesh,
        compiler_params=pltpu.CompilerParams(
            needs_layout_passes=False, use_tc_tiling_on_sc=False),
        scratch_types=[
            pltpu.VMEM((B // NS // 128, 128), jnp.int32),
            pltpu.VMEM((B // NS // 128, 128), jnp.int32),
            pltpu.VMEM((B // NS // 128, 128), jnp.int32),
            pltpu.VMEM((B // NS // 128, 128), jnp.int32),
            pltpu.VMEM((B // (NC * NS) // 128, 128), jnp.int32),
            pltpu.VMEM((B // (NC * NS) // 128, 128), jnp.int32),
            pltpu.VMEM((B * L // (NC * NS) // 128, 128), jnp.int32),
            pltpu.VMEM((B * L // (NC * NS) // 128, 128), jnp.int32),
            pltpu.VMEM((B * L // (NC * NS) // 128, 128), jnp.float32),
            pltpu.VMEM((8192,), jnp.int32),
            pltpu.VMEM_SHARED((WSZ,), jnp.int32),
            pltpu.SemaphoreType.DMA,
            pltpu.SemaphoreType.DMA,
        ],
    )(_scatter_body)

    mem_rm = _to_rowmajor(mem.T)
    val_rm = _to_rowmajor(val.T)
    out_ref = jax.new_ref(jnp.reshape(mem_rm, (M * D,)))
    scatter(out_ref, idx, jnp.reshape(val_rm, (B * D,)))
    return _to_native(jnp.reshape(out_ref[...], (M, D))).T


# speedup vs baseline: 2.7664x; 2.7664x over previous
import functools

import jax
import jax.numpy as jnp
from jax import lax
from jax.experimental import pallas as pl
from jax.experimental.pallas import tpu as pltpu
from jax.experimental.pallas import tpu_sc as plsc

L = 16
NC = 2
NS = 16
WSZ = 1 << 20
PBITS = 14


def _t_body(in_ref, out_ref):
    out_ref[...] = in_ref[...].T


def _to_rowmajor(xT, bc):
    _, N = xT.shape
    grid = (N + bc - 1) // bc
    return pl.pallas_call(
        _t_body,
        grid=(grid,),
        in_specs=[pl.BlockSpec((L, bc), lambda i: (0, i))],
        out_specs=pl.BlockSpec((bc, L), lambda i: (i, 0)),
        out_shape=jax.ShapeDtypeStruct((N, L), jnp.float32),
    )(xT)


def _to_native(x, bc):
    N, _ = x.shape
    grid = (N + bc - 1) // bc
    return pl.pallas_call(
        _t_body,
        grid=(grid,),
        in_specs=[pl.BlockSpec((bc, L), lambda i: (i, 0))],
        out_specs=pl.BlockSpec((L, bc), lambda i: (0, i)),
        out_shape=jax.ShapeDtypeStruct((L, N), jnp.float32),
    )(x)


def _scatter_body(out_ref, idx_ref, val_ref,
                  idx_d, sidx, candv, wbuf, idx_s, wsel, vrows, zbuf,
                  w_tab, sem0, sem1):
    c = lax.axis_index("c")
    s = lax.axis_index("s")
    M = out_ref.shape[0]
    B = idx_ref.shape[0]
    dpw = B // NS
    spw = B // (NC * NS)
    drows = dpw // 128
    srows = spw // 128
    iota = lax.iota(jnp.int32, L)

    def _zfill(i, _):
        zbuf[pl.ds(i * L, L)] = jnp.zeros((L,), jnp.int32)
        return 0
    lax.fori_loop(0, zbuf.shape[0] // L, _zfill, 0)
    zpw = WSZ // NS
    nz = zpw // zbuf.shape[0]
    zd = [pltpu.async_copy(zbuf, w_tab.at[pl.ds(s * zpw + t * zbuf.shape[0],
                                                zbuf.shape[0])], sem0)
          for t in range(nz)]
    dbase = s * dpw
    ld = [pltpu.async_copy(idx_ref.at[pl.ds(dbase + j * 128, 128)],
                           idx_d.at[j], sem1) for j in range(drows)]
    for d in zd + ld:
        d.wait()
    plsc.subcore_barrier()

    dummy = (M + iota).astype(jnp.int32)
    for b in range(PBITS - 1, -1, -1):
        def _prep(g, _, b=b):
            j = g // (128 // L)
            k = g % (128 // L)
            pos = dbase + g * L + iota
            w = wbuf[j, pl.ds(k * L, L)] if b < PBITS - 1 else pos * 0
            alive = lax.shift_right_logical(pos, b + 1) == \
                lax.shift_right_logical(w, b + 1)
            writer = jnp.logical_and(
                alive, (lax.shift_right_logical(pos, b) & 1) > 0)
            pref = lax.shift_left(lax.shift_right_logical(pos, b), b)
            candv[j, pl.ds(k * L, L)] = pref
            sidx[j, pl.ds(k * L, L)] = jnp.where(
                writer, idx_d[j, pl.ds(k * L, L)], dummy)
            return 0
        lax.fori_loop(0, dpw // L, _prep, 0)
        sc = [pltpu.async_copy(candv.at[j], w_tab.at[sidx.at[j]], sem0)
              for j in range(drows)]
        for d in sc:
            d.wait()
        plsc.subcore_barrier()
        if b > 0:
            ga = [pltpu.async_copy(w_tab.at[idx_d.at[j]], wbuf.at[j], sem0)
                  for j in range(drows)]
            for d in ga:
                d.wait()
            plsc.subcore_barrier()

    sbase = c * (NS * spw) + s * spw
    li = [pltpu.async_copy(idx_ref.at[pl.ds(sbase + j * 128, 128)],
                           idx_s.at[j], sem1) for j in range(srows)]
    for d in li:
        d.wait()
    gw = [pltpu.async_copy(w_tab.at[idx_s.at[j]], wsel.at[j], sem0)
          for j in range(srows)]
    for d in gw:
        d.wait()
    gv = [pltpu.async_copy(val_ref.at[wsel.at[j]],
                           vrows.at[pl.ds(j * 128, 128)], sem1)
          for j in range(srows)]
    for d in gv:
        d.wait()
    st = [pltpu.async_copy(vrows.at[pl.ds(j * 128, 128)],
                           out_ref.at[idx_s.at[j]], sem0)
          for j in range(srows)]
    for d in st:
        d.wait()


def kernel(mem, idx, val):
    M, D = mem.shape
    B = idx.shape[0]
    assert D == L and B == 1 << PBITS and M + L <= WSZ

    mesh = plsc.VectorSubcoreMesh(core_axis_name="c", subcore_axis_name="s")
    scatter = functools.partial(
        pl.kernel,
        out_type=(),
        mesh=mesh,
        compiler_params=pltpu.CompilerParams(
            needs_layout_passes=False, use_tc_tiling_on_sc=False),
        scratch_types=[
            pltpu.VMEM((B // NS // 128, 128), jnp.int32),
            pltpu.VMEM((B // NS // 128, 128), jnp.int32),
            pltpu.VMEM((B // NS // 128, 128), jnp.int32),
            pltpu.VMEM((B // NS // 128, 128), jnp.int32),
            pltpu.VMEM((B // (NC * NS) // 128, 128), jnp.int32),
            pltpu.VMEM((B // (NC * NS) // 128, 128), jnp.int32),
            pltpu.VMEM((B // (NC * NS), D), jnp.float32),
            pltpu.VMEM((8192,), jnp.int32),
            pltpu.VMEM_SHARED((WSZ,), jnp.int32),
            pltpu.SemaphoreType.DMA,
            pltpu.SemaphoreType.DMA,
        ],
    )(_scatter_body)

    mem_rm = _to_rowmajor(mem.T, 4096)
    val_rm = _to_rowmajor(val.T, 4096)
    out_ref = jax.new_ref(mem_rm)
    scatter(out_ref, idx, val_rm)
    return _to_native(out_ref[...], 4096).T


# speedup vs baseline: 2.7848x; 1.0067x over previous
import functools

import jax
import jax.numpy as jnp
from jax import lax
from jax.experimental import pallas as pl
from jax.experimental.pallas import tpu as pltpu
from jax.experimental.pallas import tpu_sc as plsc

L = 16
NC = 2
NS = 16
WSZ = 1 << 20
PBITS = 14


def _t_fwd_body(in_ref, out_ref):
    eye = jnp.eye(L, dtype=jnp.float32)
    out_ref[...] = lax.dot_general(
        in_ref[...], eye, (((0,), (0,)), ((), ())),
        preferred_element_type=jnp.float32)


def _t_bwd_body(in_ref, out_ref):
    eye = jnp.eye(L, dtype=jnp.float32)
    out_ref[...] = lax.dot_general(
        eye, in_ref[...], (((0,), (1,)), ((), ())),
        preferred_element_type=jnp.float32)


def _to_rowmajor(xT, bc):
    _, N = xT.shape
    grid = (N + bc - 1) // bc
    return pl.pallas_call(
        _t_fwd_body,
        grid=(grid,),
        in_specs=[pl.BlockSpec((L, bc), lambda i: (0, i))],
        out_specs=pl.BlockSpec((bc, L), lambda i: (i, 0)),
        out_shape=jax.ShapeDtypeStruct((N, L), jnp.float32),
    )(xT)


def _to_native(x, bc):
    N, _ = x.shape
    grid = (N + bc - 1) // bc
    return pl.pallas_call(
        _t_bwd_body,
        grid=(grid,),
        in_specs=[pl.BlockSpec((bc, L), lambda i: (i, 0))],
        out_specs=pl.BlockSpec((L, bc), lambda i: (0, i)),
        out_shape=jax.ShapeDtypeStruct((L, N), jnp.float32),
    )(x)


def _scatter_body(out_ref, idx_ref, val_ref,
                  idx_d, sidx, candv, wbuf, idx_s, wsel, vrows, zbuf,
                  w_tab, sem0, sem1):
    c = lax.axis_index("c")
    s = lax.axis_index("s")
    M = out_ref.shape[0]
    B = idx_ref.shape[0]
    dpw = B // NS
    spw = B // (NC * NS)
    drows = dpw // 128
    srows = spw // 128
    iota = lax.iota(jnp.int32, L)

    def _zfill(i, _):
        zbuf[pl.ds(i * L, L)] = jnp.zeros((L,), jnp.int32)
        return 0
    lax.fori_loop(0, zbuf.shape[0] // L, _zfill, 0)
    zpw = WSZ // NS
    nz = zpw // zbuf.shape[0]
    zd = [pltpu.async_copy(zbuf, w_tab.at[pl.ds(s * zpw + t * zbuf.shape[0],
                                                zbuf.shape[0])], sem0)
          for t in range(nz)]
    dbase = s * dpw
    ld = [pltpu.async_copy(idx_ref.at[pl.ds(dbase + j * 128, 128)],
                           idx_d.at[j], sem1) for j in range(drows)]
    for d in zd + ld:
        d.wait()
    plsc.subcore_barrier()

    dummy = (M + iota).astype(jnp.int32)
    for b in range(PBITS - 1, -1, -1):
        def _prep(g, _, b=b):
            j = g // (128 // L)
            k = g % (128 // L)
            pos = dbase + g * L + iota
            w = wbuf[j, pl.ds(k * L, L)] if b < PBITS - 1 else pos * 0
            alive = lax.shift_right_logical(pos, b + 1) == \
                lax.shift_right_logical(w, b + 1)
            writer = jnp.logical_and(
                alive, (lax.shift_right_logical(pos, b) & 1) > 0)
            pref = lax.shift_left(lax.shift_right_logical(pos, b), b)
            candv[j, pl.ds(k * L, L)] = pref
            sidx[j, pl.ds(k * L, L)] = jnp.where(
                writer, idx_d[j, pl.ds(k * L, L)], dummy)
            return 0
        lax.fori_loop(0, dpw // L, _prep, 0)
        sc = [pltpu.async_copy(candv.at[j], w_tab.at[sidx.at[j]], sem0)
              for j in range(drows)]
        for d in sc:
            d.wait()
        plsc.subcore_barrier()
        if b > 0:
            ga = [pltpu.async_copy(w_tab.at[idx_d.at[j]], wbuf.at[j], sem0)
                  for j in range(drows)]
            for d in ga:
                d.wait()
            plsc.subcore_barrier()

    sbase = c * (NS * spw) + s * spw
    li = [pltpu.async_copy(idx_ref.at[pl.ds(sbase + j * 128, 128)],
                           idx_s.at[j], sem1) for j in range(srows)]
    for d in li:
        d.wait()
    gw = [pltpu.async_copy(w_tab.at[idx_s.at[j]], wsel.at[j], sem0)
          for j in range(srows)]
    for d in gw:
        d.wait()
    gv = [pltpu.async_copy(val_ref.at[wsel.at[j]],
                           vrows.at[pl.ds(j * 128, 128)], sem1)
          for j in range(srows)]
    for d in gv:
        d.wait()
    st = [pltpu.async_copy(vrows.at[pl.ds(j * 128, 128)],
                           out_ref.at[idx_s.at[j]], sem0)
          for j in range(srows)]
    for d in st:
        d.wait()


def kernel(mem, idx, val):
    M, D = mem.shape
    B = idx.shape[0]
    assert D == L and B == 1 << PBITS and M + L <= WSZ

    mesh = plsc.VectorSubcoreMesh(core_axis_name="c", subcore_axis_name="s")
    scatter = functools.partial(
        pl.kernel,
        out_type=(),
        mesh=mesh,
        compiler_params=pltpu.CompilerParams(
            needs_layout_passes=False, use_tc_tiling_on_sc=False),
        scratch_types=[
            pltpu.VMEM((B // NS // 128, 128), jnp.int32),
            pltpu.VMEM((B // NS // 128, 128), jnp.int32),
            pltpu.VMEM((B // NS // 128, 128), jnp.int32),
            pltpu.VMEM((B // NS // 128, 128), jnp.int32),
            pltpu.VMEM((B // (NC * NS) // 128, 128), jnp.int32),
            pltpu.VMEM((B // (NC * NS) // 128, 128), jnp.int32),
            pltpu.VMEM((B // (NC * NS), D), jnp.float32),
            pltpu.VMEM((8192,), jnp.int32),
            pltpu.VMEM_SHARED((WSZ,), jnp.int32),
            pltpu.SemaphoreType.DMA,
            pltpu.SemaphoreType.DMA,
        ],
    )(_scatter_body)

    mem_rm = _to_rowmajor(mem.T, 4096)
    val_rm = _to_rowmajor(val.T, 4096)
    out_ref = jax.new_ref(mem_rm)
    scatter(out_ref, idx, val_rm)
    return _to_native(out_ref[...], 4096).T
